# baseline (speedup 1.0000x reference)
import jax
import jax.numpy as jnp
from jax import lax
from jax.experimental import pallas as pl
from jax.experimental.pallas import tpu as pltpu
from jax.experimental.pallas import tpu_sc as plsc

NCLS = 150
HROWS = 152
HCOLS = 256
NW = 32
NPIX = 8 * 512 * 512
PER_W = NPIX // NW
CHUNK = 8192
NCHUNK = PER_W // CHUNK
LANES = 16
NGROUP = 10
NSLAB = HROWS // 8


ROWS_PER_CHUNK = 16


def _hist_body(preds_hbm, truths_hbm, out_hbm, pbuf, tbuf, hist, psem, tsem):
    cid = lax.axis_index("c")
    sid = lax.axis_index("s")
    wid = sid * 2 + cid
    img = lax.shift_right_logical(wid, 2)
    rbase = lax.bitwise_and(wid, 3) * 128

    zeros = jnp.zeros((LANES,), jnp.int32)
    ones = jnp.ones((LANES,), jnp.int32)

    def zero_row(i, _):
        for u in range(HCOLS // LANES):
            hist[i, pl.ds(u * LANES, LANES)] = zeros
        return 0

    lax.fori_loop(0, HROWS, zero_row, 0)

    def start_chunk(c, slot):
        r = pl.multiple_of(rbase + c * ROWS_PER_CHUNK, ROWS_PER_CHUNK)
        pltpu.make_async_copy(
            preds_hbm.at[img, pl.ds(r, ROWS_PER_CHUNK)], pbuf.at[slot], psem.at[slot]
        ).start()
        pltpu.make_async_copy(
            truths_hbm.at[img, pl.ds(r, ROWS_PER_CHUNK)], tbuf.at[slot], tsem.at[slot]
        ).start()

    def wait_chunk(c, slot):
        r = pl.multiple_of(rbase + c * ROWS_PER_CHUNK, ROWS_PER_CHUNK)
        pltpu.make_async_copy(
            preds_hbm.at[img, pl.ds(r, ROWS_PER_CHUNK)], pbuf.at[slot], psem.at[slot]
        ).wait()
        pltpu.make_async_copy(
            truths_hbm.at[img, pl.ds(r, ROWS_PER_CHUNK)], tbuf.at[slot], tsem.at[slot]
        ).wait()

    UNROLL = 8

    start_chunk(0, 0)
    for c in range(NCHUNK):
        slot = c % 2
        wait_chunk(c, slot)
        if c + 1 < NCHUNK:
            start_chunk(c + 1, 1 - slot)

        def vec_body(i, _):
            rr = lax.shift_right_logical(i, 2)
            jb = lax.bitwise_and(i, 3) * 8
            for u in range(UNROLL):
                col = (jb + u) * LANES
                p = pbuf[slot, rr, pl.ds(col, LANES)]
                t = tbuf[slot, rr, pl.ds(col, LANES)]
                plsc.addupdate_scatter(hist, [t, p], ones)
            return 0

        lax.fori_loop(0, ROWS_PER_CHUNK * 4, vec_body, 0)

    pltpu.sync_copy(hist, out_hbm.at[wid])


def _metrics_body(part_hbm, out_hbm, inslab, cmbuf, rsbuf, obuf, ksem, cmspm):
    cid = lax.axis_index("c")
    sid = lax.axis_index("s")

    zeros = jnp.zeros((LANES,), jnp.int32)
    iota = lax.iota(jnp.int32, LANES)

    @pl.when(cid == 0)
    def _():
        for rep in range(2):
            s = sid + 16 * rep

            @pl.when(s < NSLAB)
            def _():
                r0 = pl.multiple_of(s * 8, 8)
                for k in range(NW):
                    pltpu.make_async_copy(
                        part_hbm.at[k, pl.ds(r0, 8)], inslab.at[k], ksem
                    ).start()
                for k in range(NW):
                    pltpu.make_async_copy(
                        part_hbm.at[k, pl.ds(r0, 8)], inslab.at[k], ksem
                    ).wait()

                def red_k(k, _):
                    for rr in range(8):
                        for u in range(HCOLS // LANES):
                            c = u * LANES
                            inslab[0, rr, pl.ds(c, LANES)] = (
                                inslab[0, rr, pl.ds(c, LANES)]
                                + inslab[k, rr, pl.ds(c, LANES)]
                            )
                    return 0

                lax.fori_loop(1, NW, red_k, 0)
                pltpu.sync_copy(inslab.at[0], cmspm.at[pl.ds(r0, 8)])

    plsc.subcore_barrier()

    @pl.when(jnp.logical_and(cid == 0, sid == 0))
    def _():
        pltpu.sync_copy(cmspm, cmbuf)

        for g in range(NGROUP):
            rsbuf[pl.ds(g * LANES, LANES)] = zeros

        def t_body(t, cs):
            acc = zeros
            new_cs = []
            for g in range(NGROUP):
                v = cmbuf[t, pl.ds(g * LANES, LANES)]
                new_cs.append(cs[g] + v)
                acc = acc + v
            tvec = jnp.full((LANES,), t, jnp.int32)
            plsc.addupdate_scatter(rsbuf, [tvec], acc)
            return tuple(new_cs)

        cs_fin = lax.fori_loop(0, NCLS, t_body, (zeros,) * NGROUP)

        fz = jnp.zeros((LANES,), jnp.float32)
        one_f = jnp.ones((LANES,), jnp.float32)
        nex_a = fz
        tr_a = fz
        tot_a = fz
        caa_a = fz
        iou_a = fz
        for g in range(NGROUP):
            lanecls = iota + (g * LANES)
            valid = lanecls < NCLS
            row = jnp.minimum(lanecls, HROWS - 1)
            d = plsc.load_gather(cmbuf, [row, lanecls]).astype(jnp.float32)
            cs = cs_fin[g].astype(jnp.float32)
            rs = rsbuf[pl.ds(g * LANES, LANES)].astype(jnp.float32)
            exist = jnp.logical_and(cs > 0.0, valid)
            nex_a = nex_a + jnp.where(exist, one_f, fz)
            tr_a = tr_a + jnp.where(valid, d, fz)
            tot_a = tot_a + jnp.where(valid, rs, fz)
            safe_cs = jnp.where(exist, cs, one_f)
            caa_a = caa_a + jnp.where(exist, d / safe_cs, fz)
            safe_den = jnp.where(exist, cs + rs - d, one_f)
            iou_a = iou_a + jnp.where(exist, d / safe_den, fz)

        nex = jnp.full((LANES,), lax.reduce_sum(nex_a, axes=(0,)), jnp.float32)
        tr = jnp.full((LANES,), lax.reduce_sum(tr_a, axes=(0,)), jnp.float32)
        tot = jnp.full((LANES,), lax.reduce_sum(tot_a, axes=(0,)), jnp.float32)
        caa_s = jnp.full((LANES,), lax.reduce_sum(caa_a, axes=(0,)), jnp.float32)
        iou_s = jnp.full((LANES,), lax.reduce_sum(iou_a, axes=(0,)), jnp.float32)

        res = (
            jnp.where(iota == 0, iou_s / nex, fz)
            + jnp.where(iota == 1, tr / tot, fz)
            + jnp.where(iota == 2, caa_s / nex, fz)
        )
        obuf[pl.ds(0, LANES)] = res
        pltpu.sync_copy(obuf, out_hbm)


@jax.jit
def _run(preds, truths):
    mesh = plsc.VectorSubcoreMesh(core_axis_name="c", subcore_axis_name="s")
    part = pl.kernel(
        _hist_body,
        out_type=jax.ShapeDtypeStruct((NW, HROWS, HCOLS), jnp.int32),
        mesh=mesh,
        compiler_params=pltpu.CompilerParams(
            needs_layout_passes=False, use_tc_tiling_on_sc=True
        ),
        scratch_types=[
            pltpu.VMEM((2, ROWS_PER_CHUNK, 512), jnp.int32),
            pltpu.VMEM((2, ROWS_PER_CHUNK, 512), jnp.int32),
            pltpu.VMEM((HROWS, HCOLS), jnp.int32),
            pltpu.SemaphoreType.DMA((2,)),
            pltpu.SemaphoreType.DMA((2,)),
        ],
    )(preds, truths)

    out16 = pl.kernel(
        _metrics_body,
        out_type=jax.ShapeDtypeStruct((LANES,), jnp.float32),
        mesh=mesh,
        compiler_params=pltpu.CompilerParams(needs_layout_passes=False),
        scratch_types=[
            pltpu.VMEM((NW, 8, HCOLS), jnp.int32),
            pltpu.VMEM((HROWS, HCOLS), jnp.int32),
            pltpu.VMEM((NGROUP * LANES,), jnp.int32),
            pltpu.VMEM((LANES,), jnp.float32),
            pltpu.SemaphoreType.DMA,
            pltpu.VMEM_SHARED((HROWS, HCOLS), jnp.int32),
        ],
    )(part)
    return out16


def kernel(preds, truths):
    out16 = _run(preds, truths)
    return out16[:3]


# speedup vs baseline: 1.3736x; 1.3736x over previous
import jax
import jax.numpy as jnp
from jax import lax
from jax.experimental import pallas as pl
from jax.experimental.pallas import tpu as pltpu
from jax.experimental.pallas import tpu_sc as plsc

NCLS = 150
HROWS = 152
HCOLS = 256
NW = 32
NPIX = 8 * 512 * 512
PER_W = NPIX // NW
CHUNK = 8192
NCHUNK = PER_W // CHUNK
LANES = 16
NGROUP = 10
NSLAB = HROWS // 8


ROWS_PER_CHUNK = 16


def _hist_body(preds_hbm, truths_hbm, out_hbm, pbuf, tbuf, hist, psem, tsem):
    cid = lax.axis_index("c")
    sid = lax.axis_index("s")
    wid = sid * 2 + cid
    img = lax.shift_right_logical(wid, 2)
    rbase = lax.bitwise_and(wid, 3) * 128

    zeros = jnp.zeros((LANES,), jnp.int32)
    ones = jnp.ones((LANES,), jnp.int32)

    def zero_row(i, _):
        for u in range(HCOLS // LANES):
            hist[i, pl.ds(u * LANES, LANES)] = zeros
        return 0

    lax.fori_loop(0, HROWS, zero_row, 0)

    def start_chunk(c, slot):
        r = pl.multiple_of(rbase + c * ROWS_PER_CHUNK, ROWS_PER_CHUNK)
        pltpu.make_async_copy(
            preds_hbm.at[img, pl.ds(r, ROWS_PER_CHUNK)], pbuf.at[slot], psem.at[slot]
        ).start()
        pltpu.make_async_copy(
            truths_hbm.at[img, pl.ds(r, ROWS_PER_CHUNK)], tbuf.at[slot], tsem.at[slot]
        ).start()

    def wait_chunk(c, slot):
        r = pl.multiple_of(rbase + c * ROWS_PER_CHUNK, ROWS_PER_CHUNK)
        pltpu.make_async_copy(
            preds_hbm.at[img, pl.ds(r, ROWS_PER_CHUNK)], pbuf.at[slot], psem.at[slot]
        ).wait()
        pltpu.make_async_copy(
            truths_hbm.at[img, pl.ds(r, ROWS_PER_CHUNK)], tbuf.at[slot], tsem.at[slot]
        ).wait()

    UNROLL = 8

    start_chunk(0, 0)
    for c in range(NCHUNK):
        slot = c % 2
        wait_chunk(c, slot)
        if c + 1 < NCHUNK:
            start_chunk(c + 1, 1 - slot)

        def vec_body(i, _):
            rr = lax.shift_right_logical(i, 2)
            jb = lax.bitwise_and(i, 3) * 8
            for u in range(UNROLL):
                col = (jb + u) * LANES
                p = pbuf[slot, rr, pl.ds(col, LANES)]
                t = tbuf[slot, rr, pl.ds(col, LANES)]
                plsc.addupdate_scatter(hist, [t, p], ones)
            return 0

        lax.fori_loop(0, ROWS_PER_CHUNK * 4, vec_body, 0)

    pltpu.sync_copy(hist, out_hbm.at[wid])


CPW = 1024
NCMCHUNK = HROWS * HCOLS // CPW


def _reduce_body(part_hbm, cm_hbm, islab, obuf, ksem):
    cid = lax.axis_index("c")
    sid = lax.axis_index("s")
    wid = sid * 2 + cid

    zeros = jnp.zeros((LANES,), jnp.int32)

    for rep in range(2):
        ch = wid + NW * rep

        @pl.when(ch < NCMCHUNK)
        def _():
            r0 = pl.multiple_of(lax.shift_right_logical(ch, 1) * 8, 8)
            c0 = pl.multiple_of(lax.bitwise_and(ch, 1) * 128, 128)
            for k in range(NW):
                pltpu.make_async_copy(
                    part_hbm.at[k, pl.ds(r0, 8), pl.ds(c0, 128)], islab.at[k], ksem
                ).start()
            for k in range(NW):
                pltpu.make_async_copy(
                    part_hbm.at[k, pl.ds(r0, 8), pl.ds(c0, 128)], islab.at[k], ksem
                ).wait()

            NG = 32
            for p in range(2):

                def kbody(k, accs):
                    new = []
                    for g in range(NG):
                        gg = p * NG + g
                        rr = gg // 8
                        cc = (gg % 8) * LANES
                        new.append(accs[g] + islab[k, rr, pl.ds(cc, LANES)])
                    return tuple(new)

                accs = lax.fori_loop(0, NW, kbody, (zeros,) * NG)
                for g in range(NG):
                    gg = p * NG + g
                    rr = gg // 8
                    cc = (gg % 8) * LANES
                    obuf[rr, pl.ds(cc, LANES)] = accs[g]

            pltpu.sync_copy(obuf, cm_hbm.at[pl.ds(r0, 8), pl.ds(c0, 128)])


def _metrics_body(cm_hbm, out_hbm, cmbuf, rsbuf, obuf):
    cid = lax.axis_index("c")
    sid = lax.axis_index("s")

    zeros = jnp.zeros((LANES,), jnp.int32)
    iota = lax.iota(jnp.int32, LANES)

    @pl.when(jnp.logical_and(cid == 0, sid == 0))
    def _():
        pltpu.sync_copy(cm_hbm, cmbuf)

        for g in range(NGROUP):
            rsbuf[pl.ds(g * LANES, LANES)] = zeros

        def t_body(t, cs):
            acc = zeros
            new_cs = []
            for g in range(NGROUP):
                v = cmbuf[t, pl.ds(g * LANES, LANES)]
                new_cs.append(cs[g] + v)
                acc = acc + v
            tvec = jnp.full((LANES,), t, jnp.int32)
            plsc.addupdate_scatter(rsbuf, [tvec], acc)
            return tuple(new_cs)

        cs_fin = lax.fori_loop(0, NCLS, t_body, (zeros,) * NGROUP)

        fz = jnp.zeros((LANES,), jnp.float32)
        one_f = jnp.ones((LANES,), jnp.float32)
        nex_a = fz
        tr_a = fz
        tot_a = fz
        caa_a = fz
        iou_a = fz
        for g in range(NGROUP):
            lanecls = iota + (g * LANES)
            valid = lanecls < NCLS
            row = jnp.minimum(lanecls, HROWS - 1)
            d = plsc.load_gather(cmbuf, [row, lanecls]).astype(jnp.float32)
            cs = cs_fin[g].astype(jnp.float32)
            rs = rsbuf[pl.ds(g * LANES, LANES)].astype(jnp.float32)
            exist = jnp.logical_and(cs > 0.0, valid)
            nex_a = nex_a + jnp.where(exist, one_f, fz)
            tr_a = tr_a + jnp.where(valid, d, fz)
            tot_a = tot_a + jnp.where(valid, rs, fz)
            safe_cs = jnp.where(exist, cs, one_f)
            caa_a = caa_a + jnp.where(exist, d / safe_cs, fz)
            safe_den = jnp.where(exist, cs + rs - d, one_f)
            iou_a = iou_a + jnp.where(exist, d / safe_den, fz)

        nex = jnp.full((LANES,), lax.reduce_sum(nex_a, axes=(0,)), jnp.float32)
        tr = jnp.full((LANES,), lax.reduce_sum(tr_a, axes=(0,)), jnp.float32)
        tot = jnp.full((LANES,), lax.reduce_sum(tot_a, axes=(0,)), jnp.float32)
        caa_s = jnp.full((LANES,), lax.reduce_sum(caa_a, axes=(0,)), jnp.float32)
        iou_s = jnp.full((LANES,), lax.reduce_sum(iou_a, axes=(0,)), jnp.float32)

        res = (
            jnp.where(iota == 0, iou_s / nex, fz)
            + jnp.where(iota == 1, tr / tot, fz)
            + jnp.where(iota == 2, caa_s / nex, fz)
        )
        obuf[pl.ds(0, LANES)] = res
        pltpu.sync_copy(obuf, out_hbm)


@jax.jit
def _run(preds, truths):
    mesh = plsc.VectorSubcoreMesh(core_axis_name="c", subcore_axis_name="s")
    part = pl.kernel(
        _hist_body,
        out_type=jax.ShapeDtypeStruct((NW, HROWS, HCOLS), jnp.int32),
        mesh=mesh,
        compiler_params=pltpu.CompilerParams(
            needs_layout_passes=False, use_tc_tiling_on_sc=True
        ),
        scratch_types=[
            pltpu.VMEM((2, ROWS_PER_CHUNK, 512), jnp.int32),
            pltpu.VMEM((2, ROWS_PER_CHUNK, 512), jnp.int32),
            pltpu.VMEM((HROWS, HCOLS), jnp.int32),
            pltpu.SemaphoreType.DMA((2,)),
            pltpu.SemaphoreType.DMA((2,)),
        ],
    )(preds, truths)

    cm = pl.kernel(
        _reduce_body,
        out_type=jax.ShapeDtypeStruct((HROWS, HCOLS), jnp.int32),
        mesh=mesh,
        compiler_params=pltpu.CompilerParams(needs_layout_passes=False),
        scratch_types=[
            pltpu.VMEM((NW, 8, 128), jnp.int32),
            pltpu.VMEM((8, 128), jnp.int32),
            pltpu.SemaphoreType.DMA,
        ],
    )(part)

    out16 = pl.kernel(
        _metrics_body,
        out_type=jax.ShapeDtypeStruct((LANES,), jnp.float32),
        mesh=mesh,
        compiler_params=pltpu.CompilerParams(needs_layout_passes=False),
        scratch_types=[
            pltpu.VMEM((HROWS, HCOLS), jnp.int32),
            pltpu.VMEM((NGROUP * LANES,), jnp.int32),
            pltpu.VMEM((LANES,), jnp.float32),
        ],
    )(cm)
    return out16


def kernel(preds, truths):
    out16 = _run(preds, truths)
    return out16[:3]


# speedup vs baseline: 2.1270x; 1.5485x over previous
import jax
import jax.numpy as jnp
from jax import lax
from jax.experimental import pallas as pl
from jax.experimental.pallas import tpu as pltpu
from jax.experimental.pallas import tpu_sc as plsc

NCLS = 150
HROWS = 152
HCOLS = 256
NW = 32
NPIX = 8 * 512 * 512
PER_W = NPIX // NW
CHUNK = 8192
NCHUNK = PER_W // CHUNK
LANES = 16
NGROUP = 10
NSLAB = HROWS // 8


ROWS_PER_CHUNK = 16


def _hist_body(preds_hbm, truths_hbm, out_hbm, pbuf, tbuf, hist, psem, tsem):
    cid = lax.axis_index("c")
    sid = lax.axis_index("s")
    wid = sid * 2 + cid
    img = lax.shift_right_logical(wid, 2)
    rbase = lax.bitwise_and(wid, 3) * 128

    zeros = jnp.zeros((LANES,), jnp.int32)
    ones = jnp.ones((LANES,), jnp.int32)

    @plsc.parallel_loop(0, HROWS, 1, unroll=4)
    def _(i):
        for u in range(HCOLS // LANES):
            hist[i, pl.ds(u * LANES, LANES)] = zeros

    def start_chunk(c, slot):
        r = pl.multiple_of(rbase + c * ROWS_PER_CHUNK, ROWS_PER_CHUNK)
        pltpu.make_async_copy(
            preds_hbm.at[img, pl.ds(r, ROWS_PER_CHUNK)], pbuf.at[slot], psem.at[slot]
        ).start()
        pltpu.make_async_copy(
            truths_hbm.at[img, pl.ds(r, ROWS_PER_CHUNK)], tbuf.at[slot], tsem.at[slot]
        ).start()

    def wait_chunk(c, slot):
        r = pl.multiple_of(rbase + c * ROWS_PER_CHUNK, ROWS_PER_CHUNK)
        pltpu.make_async_copy(
            preds_hbm.at[img, pl.ds(r, ROWS_PER_CHUNK)], pbuf.at[slot], psem.at[slot]
        ).wait()
        pltpu.make_async_copy(
            truths_hbm.at[img, pl.ds(r, ROWS_PER_CHUNK)], tbuf.at[slot], tsem.at[slot]
        ).wait()

    UNROLL = 8

    start_chunk(0, 0)
    for c in range(NCHUNK):
        slot = c % 2
        wait_chunk(c, slot)
        if c + 1 < NCHUNK:
            start_chunk(c + 1, 1 - slot)

        @plsc.parallel_loop(0, ROWS_PER_CHUNK * 32, 1, unroll=UNROLL)
        def _(i):
            rr = lax.shift_right_logical(i, 5)
            col = lax.bitwise_and(i, 31) * LANES
            p = pbuf[slot, rr, pl.ds(col, LANES)]
            t = tbuf[slot, rr, pl.ds(col, LANES)]
            plsc.addupdate_scatter(hist, [t, p], ones)

    pltpu.sync_copy(hist, out_hbm.at[wid])


CPW = 1024
NCMCHUNK = HROWS * HCOLS // CPW


def _reduce_body(part_hbm, cm_hbm, islab, obuf, ksem):
    cid = lax.axis_index("c")
    sid = lax.axis_index("s")
    wid = sid * 2 + cid

    zeros = jnp.zeros((LANES,), jnp.int32)

    for rep in range(2):
        ch = wid + NW * rep

        @pl.when(ch < NCMCHUNK)
        def _():
            r0 = pl.multiple_of(lax.shift_right_logical(ch, 1) * 8, 8)
            c0 = pl.multiple_of(lax.bitwise_and(ch, 1) * 128, 128)
            for k in range(NW):
                pltpu.make_async_copy(
                    part_hbm.at[k, pl.ds(r0, 8), pl.ds(c0, 128)], islab.at[k], ksem
                ).start()
            for k in range(NW):
                pltpu.make_async_copy(
                    part_hbm.at[k, pl.ds(r0, 8), pl.ds(c0, 128)], islab.at[k], ksem
                ).wait()

            NG = 32
            for p in range(2):

                def kbody(k, accs):
                    new = []
                    for g in range(NG):
                        gg = p * NG + g
                        rr = gg // 8
                        cc = (gg % 8) * LANES
                        new.append(accs[g] + islab[k, rr, pl.ds(cc, LANES)])
                    return tuple(new)

                accs = lax.fori_loop(0, NW, kbody, (zeros,) * NG)
                for g in range(NG):
                    gg = p * NG + g
                    rr = gg // 8
                    cc = (gg % 8) * LANES
                    obuf[rr, pl.ds(cc, LANES)] = accs[g]

            pltpu.sync_copy(obuf, cm_hbm.at[pl.ds(r0, 8), pl.ds(c0, 128)])


def _metrics_body(cm_hbm, out_hbm, cmbuf, rsbuf, obuf):
    cid = lax.axis_index("c")
    sid = lax.axis_index("s")

    zeros = jnp.zeros((LANES,), jnp.int32)
    iota = lax.iota(jnp.int32, LANES)

    @pl.when(jnp.logical_and(cid == 0, sid == 0))
    def _():
        pltpu.sync_copy(cm_hbm, cmbuf)

        for g in range(NGROUP):
            rsbuf[pl.ds(g * LANES, LANES)] = zeros

        def t_body(t, cs):
            acc = zeros
            new_cs = []
            for g in range(NGROUP):
                v = cmbuf[t, pl.ds(g * LANES, LANES)]
                new_cs.append(cs[g] + v)
                acc = acc + v
            tvec = jnp.full((LANES,), t, jnp.int32)
            plsc.addupdate_scatter(rsbuf, [tvec], acc)
            return tuple(new_cs)

        cs_fin = lax.fori_loop(0, NCLS, t_body, (zeros,) * NGROUP)

        fz = jnp.zeros((LANES,), jnp.float32)
        one_f = jnp.ones((LANES,), jnp.float32)
        nex_a = fz
        tr_a = fz
        tot_a = fz
        caa_a = fz
        iou_a = fz
        for g in range(NGROUP):
            lanecls = iota + (g * LANES)
            valid = lanecls < NCLS
            row = jnp.minimum(lanecls, HROWS - 1)
            d = plsc.load_gather(cmbuf, [row, lanecls]).astype(jnp.float32)
            cs = cs_fin[g].astype(jnp.float32)
            rs = rsbuf[pl.ds(g * LANES, LANES)].astype(jnp.float32)
            exist = jnp.logical_and(cs > 0.0, valid)
            nex_a = nex_a + jnp.where(exist, one_f, fz)
            tr_a = tr_a + jnp.where(valid, d, fz)
            tot_a = tot_a + jnp.where(valid, rs, fz)
            safe_cs = jnp.where(exist, cs, one_f)
            caa_a = caa_a + jnp.where(exist, d / safe_cs, fz)
            safe_den = jnp.where(exist, cs + rs - d, one_f)
            iou_a = iou_a + jnp.where(exist, d / safe_den, fz)

        nex = jnp.full((LANES,), lax.reduce_sum(nex_a, axes=(0,)), jnp.float32)
        tr = jnp.full((LANES,), lax.reduce_sum(tr_a, axes=(0,)), jnp.float32)
        tot = jnp.full((LANES,), lax.reduce_sum(tot_a, axes=(0,)), jnp.float32)
        caa_s = jnp.full((LANES,), lax.reduce_sum(caa_a, axes=(0,)), jnp.float32)
        iou_s = jnp.full((LANES,), lax.reduce_sum(iou_a, axes=(0,)), jnp.float32)

        res = (
            jnp.where(iota == 0, iou_s / nex, fz)
            + jnp.where(iota == 1, tr / tot, fz)
            + jnp.where(iota == 2, caa_s / nex, fz)
        )
        obuf[pl.ds(0, LANES)] = res
        pltpu.sync_copy(obuf, out_hbm)


@jax.jit
def _run(preds, truths):
    mesh = plsc.VectorSubcoreMesh(core_axis_name="c", subcore_axis_name="s")
    part = pl.kernel(
        _hist_body,
        out_type=jax.ShapeDtypeStruct((NW, HROWS, HCOLS), jnp.int32),
        mesh=mesh,
        compiler_params=pltpu.CompilerParams(
            needs_layout_passes=False, use_tc_tiling_on_sc=True
        ),
        scratch_types=[
            pltpu.VMEM((2, ROWS_PER_CHUNK, 512), jnp.int32),
            pltpu.VMEM((2, ROWS_PER_CHUNK, 512), jnp.int32),
            pltpu.VMEM((HROWS, HCOLS), jnp.int32),
            pltpu.SemaphoreType.DMA((2,)),
            pltpu.SemaphoreType.DMA((2,)),
        ],
    )(preds, truths)

    cm = pl.kernel(
        _reduce_body,
        out_type=jax.ShapeDtypeStruct((HROWS, HCOLS), jnp.int32),
        mesh=mesh,
        compiler_params=pltpu.CompilerParams(needs_layout_passes=False),
        scratch_types=[
            pltpu.VMEM((NW, 8, 128), jnp.int32),
            pltpu.VMEM((8, 128), jnp.int32),
            pltpu.SemaphoreType.DMA,
        ],
    )(part)

    out16 = pl.kernel(
        _metrics_body,
        out_type=jax.ShapeDtypeStruct((LANES,), jnp.float32),
        mesh=mesh,
        compiler_params=pltpu.CompilerParams(needs_layout_passes=False),
        scratch_types=[
            pltpu.VMEM((HROWS, HCOLS), jnp.int32),
            pltpu.VMEM((NGROUP * LANES,), jnp.int32),
            pltpu.VMEM((LANES,), jnp.float32),
        ],
    )(cm)
    return out16


def kernel(preds, truths):
    out16 = _run(preds, truths)
    return out16[:3]
